# Initial kernel scaffold; baseline (speedup 1.0000x reference)
import functools

import jax
import jax.numpy as jnp
from jax import lax
from jax.experimental import pallas as pl
from jax.experimental.pallas import tpu as pltpu
from jax.experimental.pallas import tpu_sc as plsc

N = 10000
E = 160000
D = 256
H = 128
OUT = 10
JITTER = 1e-3

R = 1000
NB = N // R
NTILES = 16
EP = E // NTILES
K = 200
NCHUNK = EP // K
RP = N // NTILES


def _colsum_body(x_ref, out_ref):
    @pl.when(pl.program_id(0) == 0)
    def _():
        out_ref[...] = jnp.zeros_like(out_ref)

    out_ref[...] += jnp.sum(x_ref[...], axis=0, keepdims=True)


def _colsum(x):
    return pl.pallas_call(
        _colsum_body,
        grid=(NB,),
        in_specs=[pl.BlockSpec((R, D), lambda i: (i, 0))],
        out_specs=pl.BlockSpec((1, D), lambda i: (0, 0)),
        out_shape=jax.ShapeDtypeStruct((1, D), jnp.float32),
    )(x)


def _center_body(x_ref, cs_ref, a_ref, b_ref):
    xc = x_ref[...] - cs_ref[...] * (1.0 / N)
    a_ref[...] = xc[:, :H]
    b_ref[...] = xc[:, H:]


def _center_split(x, colsum):
    return pl.pallas_call(
        _center_body,
        grid=(NB,),
        in_specs=[
            pl.BlockSpec((R, D), lambda i: (i, 0)),
            pl.BlockSpec((1, D), lambda i: (0, 0)),
        ],
        out_specs=[
            pl.BlockSpec((R, H), lambda i: (i, 0)),
            pl.BlockSpec((R, H), lambda i: (i, 0)),
        ],
        out_shape=[
            jax.ShapeDtypeStruct((N, H), jnp.float32),
            jax.ShapeDtypeStruct((N, H), jnp.float32),
        ],
    )(x, colsum)


def _sc_body(xca, xcb, src, dst, z128, z16, ones_hbm,
             agga, aggb, deg16,
             acc_sh, deg_sh, src_v, dst_v, rows_v, ones_v, sem):
    cid = lax.axis_index("c")
    sid = lax.axis_index("s")
    row0 = sid * RP

    pltpu.sync_copy(z128, acc_sh.at[pl.ds(row0, RP)])
    pltpu.sync_copy(z16, deg_sh.at[pl.ds(row0, RP)])
    pltpu.sync_copy(ones_hbm, ones_v)
    plsc.subcore_barrier()

    def run(xc_hbm, agg_hbm, do_deg):
        def chunk(i, carry):
            eb = sid * EP + i * K
            pltpu.sync_copy(src.at[pl.ds(eb, K)], src_v)
            pltpu.sync_copy(dst.at[pl.ds(eb, K)], dst_v)
            pltpu.async_copy(xc_hbm.at[src_v], rows_v, sem).wait()
            pltpu.sync_copy(rows_v, acc_sh.at[dst_v], add=True)
            if do_deg:
                pltpu.sync_copy(ones_v, deg_sh.at[dst_v], add=True)
            return carry

        lax.fori_loop(0, NCHUNK, chunk, 0)
        plsc.subcore_barrier()
        pltpu.sync_copy(acc_sh.at[pl.ds(row0, RP)], agg_hbm.at[pl.ds(row0, RP)])
        if do_deg:
            pltpu.sync_copy(deg_sh.at[pl.ds(row0, RP)], deg16.at[pl.ds(row0, RP)])

    @pl.when(cid == 0)
    def _():
        run(xca, agga, True)

    @pl.when(cid == 1)
    def _():
        run(xcb, aggb, False)


def _sc_aggregate(xca, xcb, src, dst):
    z128 = jnp.zeros((RP, H), jnp.float32)
    z16 = jnp.zeros((RP, 16), jnp.float32)
    ones16 = jnp.ones((K, 16), jnp.float32)
    mesh = plsc.VectorSubcoreMesh(core_axis_name="c", subcore_axis_name="s")
    f = pl.kernel(
        _sc_body,
        out_type=[
            jax.ShapeDtypeStruct((N, H), jnp.float32),
            jax.ShapeDtypeStruct((N, H), jnp.float32),
            jax.ShapeDtypeStruct((N, 16), jnp.float32),
        ],
        mesh=mesh,
        scratch_types=[
            pltpu.VMEM_SHARED((N, H), jnp.float32),
            pltpu.VMEM_SHARED((N, 16), jnp.float32),
            pltpu.VMEM((K,), jnp.int32),
            pltpu.VMEM((K,), jnp.int32),
            pltpu.VMEM((K, H), jnp.float32),
            pltpu.VMEM((K, 16), jnp.float32),
            pltpu.SemaphoreType.DMA,
        ],
    )
    return f(xca, xcb, src, dst, z128, z16, ones16)


def _softplus(v):
    return jnp.maximum(v, 0.0) + jnp.log(1.0 + jnp.exp(-jnp.abs(v)))


def _dense_body(xca, xcb, agga, aggb, deg16,
                aWlT, aWrT, a_bc, aW1T, a_b1, aW2T, a_b2, aW3T, a_b3,
                cWlT, cWrT, c_bc, cW1T, c_b1, cW2T, c_b2, cW3T, c_b3,
                conc_ref, hsum_ref, value_ref):
    i = pl.program_id(0)
    inv = 1.0 / jnp.maximum(deg16[:, 0:1], 1.0)
    agg = jnp.concatenate([agga[...], aggb[...]], axis=1) * inv
    xc = jnp.concatenate([xca[...], xcb[...]], axis=1)

    dot = functools.partial(jnp.dot, preferred_element_type=jnp.float32)

    pre = dot(agg, aWlT[...]) + dot(xc, aWrT[...]) + a_bc[...]
    h = jnp.maximum(pre, 0.0) + xc
    h1 = jnp.maximum(dot(h, aW1T[...]) + a_b1[...], 0.0)
    h2 = jnp.maximum(dot(h1, aW2T[...]) + a_b2[...], 0.0)
    ao = dot(h2, aW3T[...]) + a_b3[...]
    conc_ref[...] = _softplus(ao) + JITTER

    prec = dot(agg, cWlT[...]) + dot(xc, cWrT[...]) + c_bc[...]
    hcb = jnp.maximum(prec, 0.0) + xc
    part = jnp.sum(hcb, axis=0, keepdims=True)

    @pl.when(i == 0)
    def _():
        hsum_ref[...] = part

    @pl.when(i > 0)
    def _():
        hsum_ref[...] += part

    @pl.when(i == NB - 1)
    def _():
        hc = hsum_ref[...]
        v1 = jnp.maximum(dot(hc, cW1T[...]) + c_b1[...], 0.0)
        v2 = jnp.maximum(dot(v1, cW2T[...]) + c_b2[...], 0.0)
        value_ref[...] = dot(v2, cW3T[...]) + c_b3[...]


def _dense(xca, xcb, agga, aggb, deg16, aw, cw):
    def full(shape):
        return pl.BlockSpec(shape, lambda i: tuple(0 for _ in shape))

    def row(w):
        return pl.BlockSpec((R, w), lambda i: (i, 0))

    wspecs = [full(w.shape) for w in aw] + [full(w.shape) for w in cw]
    return pl.pallas_call(
        _dense_body,
        grid=(NB,),
        in_specs=[row(H), row(H), row(H), row(H), row(16)] + wspecs,
        out_specs=[
            pl.BlockSpec((R, OUT), lambda i: (i, 0)),
            pl.BlockSpec((1, D), lambda i: (0, 0)),
            pl.BlockSpec((1, OUT), lambda i: (0, 0)),
        ],
        out_shape=[
            jax.ShapeDtypeStruct((N, OUT), jnp.float32),
            jax.ShapeDtypeStruct((1, D), jnp.float32),
            jax.ShapeDtypeStruct((1, OUT), jnp.float32),
        ],
    )(xca, xcb, agga, aggb, deg16, *aw, *cw)


def kernel(x, edge_index, a_Wl, a_Wr, a_bc, a_W1, a_b1, a_W2, a_b2, a_W3, a_b3,
           c_Wl, c_Wr, c_bc, c_W1, c_b1, c_W2, c_b2, c_W3, c_b3):
    src = edge_index[0]
    dst = edge_index[1]

    colsum = _colsum(x)
    xca, xcb = _center_split(x, colsum)
    agga, aggb, deg16 = _sc_aggregate(xca, xcb, src, dst)

    aw = [a_Wl.T, a_Wr.T, a_bc.reshape(1, -1), a_W1.T, a_b1.reshape(1, -1),
          a_W2.T, a_b2.reshape(1, -1), a_W3.T, a_b3.reshape(1, -1)]
    cw = [c_Wl.T, c_Wr.T, c_bc.reshape(1, -1), c_W1.T, c_b1.reshape(1, -1),
          c_W2.T, c_b2.reshape(1, -1), c_W3.T, c_b3.reshape(1, -1)]
    conc2d, _, value = _dense(xca, xcb, agga, aggb, deg16, aw, cw)
    return conc2d.reshape(-1), value.reshape(OUT)


# speedup vs baseline: 3.6032x; 3.6032x over previous
import functools

import jax
import jax.numpy as jnp
from jax import lax
from jax.experimental import pallas as pl
from jax.experimental.pallas import tpu as pltpu
from jax.experimental.pallas import tpu_sc as plsc

N = 10000
E = 160000
D = 256
H = 128
OUT = 10
JITTER = 1e-3

R = 1000
NB = N // R
NTILES = 16
EP = E // NTILES
K = 80
NCHUNK = EP // K
NPAD = 10240
RP = NPAD // NTILES
NZ = RP // K
HA = H + 16


def _colsum_body(x_ref, out_ref):
    @pl.when(pl.program_id(0) == 0)
    def _():
        out_ref[...] = jnp.zeros_like(out_ref)

    out_ref[...] += jnp.sum(x_ref[...], axis=0, keepdims=True)


def _colsum(x):
    return pl.pallas_call(
        _colsum_body,
        grid=(NB,),
        in_specs=[pl.BlockSpec((R, D), lambda i: (i, 0))],
        out_specs=pl.BlockSpec((1, D), lambda i: (0, 0)),
        out_shape=jax.ShapeDtypeStruct((1, D), jnp.float32),
    )(x)


def _center_body(x_ref, cs_ref, a_ref, b_ref):
    xc = x_ref[...] - cs_ref[...] * (1.0 / N)
    ones = jnp.ones((R, 16), jnp.float32)
    a_ref[...] = jnp.concatenate([xc[:, :H], ones], axis=1)
    b_ref[...] = jnp.concatenate([xc[:, H:], ones], axis=1)


def _center_split(x, colsum):
    return pl.pallas_call(
        _center_body,
        grid=(NB,),
        in_specs=[
            pl.BlockSpec((R, D), lambda i: (i, 0)),
            pl.BlockSpec((1, D), lambda i: (0, 0)),
        ],
        out_specs=[
            pl.BlockSpec((R, HA), lambda i: (i, 0)),
            pl.BlockSpec((R, HA), lambda i: (i, 0)),
        ],
        out_shape=[
            jax.ShapeDtypeStruct((N, HA), jnp.float32),
            jax.ShapeDtypeStruct((N, HA), jnp.float32),
        ],
    )(x, colsum)


def _sc_body(xca, xcb, src, dst, z128,
             agga, aggb,
             acc_sh, src_v, dst_v, rows_v, sem):
    cid = lax.axis_index("c")
    sid = lax.axis_index("s")
    row0 = sid * RP

    pltpu.sync_copy(z128, rows_v)
    for j in range(NZ):
        pltpu.sync_copy(rows_v, acc_sh.at[pl.ds(row0 + j * K, K)])
    plsc.subcore_barrier()

    def run(xc_hbm):
        @pl.loop(0, NCHUNK, unroll=1)
        def _chunk(i):
            eb = sid * EP + i * K
            pltpu.sync_copy(src.at[pl.ds(eb, K)], src_v)
            pltpu.sync_copy(dst.at[pl.ds(eb, K)], dst_v)
            pltpu.async_copy(xc_hbm.at[src_v], rows_v, sem).wait()
            pltpu.sync_copy(rows_v, acc_sh.at[dst_v], add=True)

    @pl.when(cid == 0)
    def _():
        run(xca)

    @pl.when(cid == 1)
    def _():
        run(xcb)

    plsc.subcore_barrier()

    def wb(out_hbm):
        @pl.loop(0, NZ, unroll=1)
        def _step(j):
            r0 = row0 + j * K
            pltpu.sync_copy(acc_sh.at[pl.ds(r0, K)], rows_v)
            pltpu.sync_copy(rows_v, out_hbm.at[pl.ds(r0, K)])

    @pl.when(cid == 0)
    def _():
        wb(agga)

    @pl.when(cid == 1)
    def _():
        wb(aggb)


def _sc_aggregate(xca, xcb, src, dst):
    z128 = jnp.zeros((K, HA), jnp.float32)
    mesh = plsc.VectorSubcoreMesh(core_axis_name="c", subcore_axis_name="s")
    f = pl.kernel(
        _sc_body,
        out_type=[
            jax.ShapeDtypeStruct((NPAD, HA), jnp.float32),
            jax.ShapeDtypeStruct((NPAD, HA), jnp.float32),
        ],
        mesh=mesh,
        scratch_types=[
            pltpu.VMEM_SHARED((NPAD, HA), jnp.float32),
            pltpu.VMEM((K,), jnp.int32),
            pltpu.VMEM((K,), jnp.int32),
            pltpu.VMEM((K, HA), jnp.float32),
            pltpu.SemaphoreType.DMA,
        ],
        compiler_params=pltpu.CompilerParams(use_tc_tiling_on_sc=False),
    )
    return f(xca, xcb, src, dst, z128)


def _softplus(v):
    return jnp.maximum(v, 0.0) + jnp.log(1.0 + jnp.exp(-jnp.abs(v)))


def _dense_body(xca, xcb, agga, aggb,
                aWlT, aWrT, a_bc, aW1T, a_b1, aW2T, a_b2, aW3T, a_b3,
                cWlT, cWrT, c_bc, cW1T, c_b1, cW2T, c_b2, cW3T, c_b3,
                conc_ref, hsum_ref, value_ref):
    i = pl.program_id(0)
    inv = 1.0 / jnp.maximum(agga[:, H:H + 1], 1.0)
    agg = jnp.concatenate([agga[:, :H], aggb[:, :H]], axis=1) * inv
    xc = jnp.concatenate([xca[:, :H], xcb[:, :H]], axis=1)

    dot = functools.partial(jnp.dot, preferred_element_type=jnp.float32)

    pre = dot(agg, aWlT[...]) + dot(xc, aWrT[...]) + a_bc[...]
    h = jnp.maximum(pre, 0.0) + xc
    h1 = jnp.maximum(dot(h, aW1T[...]) + a_b1[...], 0.0)
    h2 = jnp.maximum(dot(h1, aW2T[...]) + a_b2[...], 0.0)
    ao = dot(h2, aW3T[...]) + a_b3[...]
    conc_ref[...] = _softplus(ao) + JITTER

    prec = dot(agg, cWlT[...]) + dot(xc, cWrT[...]) + c_bc[...]
    hcb = jnp.maximum(prec, 0.0) + xc
    part = jnp.sum(hcb, axis=0, keepdims=True)

    @pl.when(i == 0)
    def _():
        hsum_ref[...] = part

    @pl.when(i > 0)
    def _():
        hsum_ref[...] += part

    @pl.when(i == NB - 1)
    def _():
        hc = hsum_ref[...]
        v1 = jnp.maximum(dot(hc, cW1T[...]) + c_b1[...], 0.0)
        v2 = jnp.maximum(dot(v1, cW2T[...]) + c_b2[...], 0.0)
        value_ref[...] = dot(v2, cW3T[...]) + c_b3[...]


def _dense(xca, xcb, agga, aggb, aw, cw):
    def full(shape):
        return pl.BlockSpec(shape, lambda i: tuple(0 for _ in shape))

    def row(w):
        return pl.BlockSpec((R, w), lambda i: (i, 0))

    wspecs = [full(w.shape) for w in aw] + [full(w.shape) for w in cw]
    return pl.pallas_call(
        _dense_body,
        grid=(NB,),
        in_specs=[row(HA), row(HA), row(HA), row(HA)] + wspecs,
        out_specs=[
            pl.BlockSpec((R, OUT), lambda i: (i, 0)),
            pl.BlockSpec((1, D), lambda i: (0, 0)),
            pl.BlockSpec((1, OUT), lambda i: (0, 0)),
        ],
        out_shape=[
            jax.ShapeDtypeStruct((N, OUT), jnp.float32),
            jax.ShapeDtypeStruct((1, D), jnp.float32),
            jax.ShapeDtypeStruct((1, OUT), jnp.float32),
        ],
    )(xca, xcb, agga, aggb, *aw, *cw)


def kernel(x, edge_index, a_Wl, a_Wr, a_bc, a_W1, a_b1, a_W2, a_b2, a_W3, a_b3,
           c_Wl, c_Wr, c_bc, c_W1, c_b1, c_W2, c_b2, c_W3, c_b3):
    src = edge_index[0]
    dst = edge_index[1]

    colsum = _colsum(x)
    xca, xcb = _center_split(x, colsum)
    agga, aggb = _sc_aggregate(xca, xcb, src, dst)

    aw = [a_Wl.T, a_Wr.T, a_bc.reshape(1, -1), a_W1.T, a_b1.reshape(1, -1),
          a_W2.T, a_b2.reshape(1, -1), a_W3.T, a_b3.reshape(1, -1)]
    cw = [c_Wl.T, c_Wr.T, c_bc.reshape(1, -1), c_W1.T, c_b1.reshape(1, -1),
          c_W2.T, c_b2.reshape(1, -1), c_W3.T, c_b3.reshape(1, -1)]
    conc2d, _, value = _dense(xca, xcb, agga, aggb, aw, cw)
    return conc2d.reshape(-1), value.reshape(OUT)
